# baseline (speedup 1.0000x reference)
import functools

import jax
import jax.numpy as jnp
from jax import lax
from jax.experimental import pallas as pl
from jax.experimental.pallas import tpu as pltpu
from jax.experimental.pallas import tpu_sc as plsc

BATCH = 16384
VOCAB = 1000000
DIM = 32
NC = 2
NS = 16
L = 16
NW = NC * NS
BPW = BATCH // NW
CHUNK = 128
NCHUNK = BPW // CHUNK
NBUF = 2


def _sc_body(didx_hbm, aidx_hbm, dflat_hbm, aflat_hbm, out_hbm,
             didx_v, aidx_v, dbuf_v, abuf_v, out_v, sems):
    wid = lax.axis_index("s") * NC + lax.axis_index("c")
    base = wid * BPW

    pltpu.sync_copy(didx_hbm.at[pl.ds(base, BPW)], didx_v)
    pltpu.sync_copy(aidx_hbm.at[pl.ds(base, BPW)], aidx_v)

    def start(c):
        b = c % NBUF
        sl = pl.ds(c * CHUNK, CHUNK)
        cps = []
        for j in range(DIM):
            comp = pl.ds(j * VOCAB, VOCAB)
            cps.append(pltpu.async_copy(
                dflat_hbm.at[comp].at[didx_v.at[sl]],
                dbuf_v.at[b, j], sems.at[b]))
            cps.append(pltpu.async_copy(
                aflat_hbm.at[comp].at[aidx_v.at[sl]],
                abuf_v.at[b, j], sems.at[b]))
        return cps

    inflight = {c: start(c) for c in range(min(NBUF, NCHUNK))}

    for c in range(NCHUNK):
        b = c % NBUF
        for cp in inflight.pop(c):
            cp.wait()

        def group(g, carry):
            sl = pl.ds(g * L, L)
            acc = jnp.zeros((L,), jnp.float32)
            for j in range(DIM):
                acc = acc + dbuf_v[b, j, sl] * abuf_v[b, j, sl]
            out_v[pl.ds(c * CHUNK + g * L, L)] = acc
            return carry

        lax.fori_loop(0, CHUNK // L, group, 0)

        if c + NBUF < NCHUNK:
            inflight[c + NBUF] = start(c + NBUF)

    pltpu.sync_copy(out_v, out_hbm.at[pl.ds(base, BPW)])


@functools.partial(
    pl.kernel,
    mesh=plsc.VectorSubcoreMesh(core_axis_name="c", subcore_axis_name="s"),
    out_type=jax.ShapeDtypeStruct((BATCH,), jnp.float32),
    scratch_types=[
        pltpu.VMEM((BPW,), jnp.int32),
        pltpu.VMEM((BPW,), jnp.int32),
        pltpu.VMEM((NBUF, DIM, CHUNK), jnp.float32),
        pltpu.VMEM((NBUF, DIM, CHUNK), jnp.float32),
        pltpu.VMEM((BPW,), jnp.float32),
        pltpu.SemaphoreType.DMA((NBUF,)),
    ],
    compiler_params=pltpu.CompilerParams(
        needs_layout_passes=False, use_tc_tiling_on_sc=False),
)
def _sc_call(didx_hbm, aidx_hbm, dflat_hbm, aflat_hbm, out_hbm,
             didx_v, aidx_v, dbuf_v, abuf_v, out_v, sems):
    _sc_body(didx_hbm, aidx_hbm, dflat_hbm, aflat_hbm, out_hbm,
             didx_v, aidx_v, dbuf_v, abuf_v, out_v, sems)


@jax.jit
def kernel(drug_idx, adr_idx, drug_embeddings, adr_embeddings):
    dflat = drug_embeddings.T.reshape(-1)
    aflat = adr_embeddings.T.reshape(-1)
    return _sc_call(drug_idx, adr_idx, dflat, aflat)


# speedup vs baseline: 5.6507x; 5.6507x over previous
import functools

import jax
import jax.numpy as jnp
from jax import lax
from jax.experimental import pallas as pl
from jax.experimental.pallas import tpu as pltpu
from jax.experimental.pallas import tpu_sc as plsc

BATCH = 16384
DIM = 32
NC = 2
NS = 16
L = 16
NW = NC * NS
BPW = BATCH // NW
CHUNK = 128
NCHUNK = BPW // CHUNK


def _sc_body(didx_hbm, aidx_hbm, dtab_hbm, atab_hbm, out_hbm,
             didx_v, aidx_v, drows_v, arows_v, out_v, sem):
    wid = lax.axis_index("s") * NC + lax.axis_index("c")

    pltpu.sync_copy(didx_hbm.at[pl.ds(wid * NCHUNK, NCHUNK)], didx_v)
    pltpu.sync_copy(aidx_hbm.at[pl.ds(wid * NCHUNK, NCHUNK)], aidx_v)

    copies = []
    for c in range(NCHUNK):
        copies.append(pltpu.async_copy(
            dtab_hbm.at[didx_v.at[c]], drows_v.at[pl.ds(c * CHUNK, CHUNK)],
            sem))
        copies.append(pltpu.async_copy(
            atab_hbm.at[aidx_v.at[c]], arows_v.at[pl.ds(c * CHUNK, CHUNK)],
            sem))
    for cp in copies:
        cp.wait()

    lane = lax.iota(jnp.int32, L)

    def group(g, carry):
        rows = g * L + lane
        acc = jnp.zeros((L,), jnp.float32)
        for j in range(DIM):
            col = jnp.full((L,), j, jnp.int32)
            dv = plsc.load_gather(drows_v, [rows, col])
            av = plsc.load_gather(arows_v, [rows, col])
            acc = acc + dv * av
        out_v[pl.ds(g * L, L)] = acc
        return carry

    lax.fori_loop(0, BPW // L, group, 0)

    pltpu.sync_copy(out_v, out_hbm.at[pl.ds(wid * BPW, BPW)])


@functools.partial(
    pl.kernel,
    mesh=plsc.VectorSubcoreMesh(core_axis_name="c", subcore_axis_name="s"),
    out_type=jax.ShapeDtypeStruct((BATCH,), jnp.float32),
    scratch_types=[
        pltpu.VMEM((NCHUNK, CHUNK), jnp.int32),
        pltpu.VMEM((NCHUNK, CHUNK), jnp.int32),
        pltpu.VMEM((BPW, DIM), jnp.float32),
        pltpu.VMEM((BPW, DIM), jnp.float32),
        pltpu.VMEM((BPW,), jnp.float32),
        pltpu.SemaphoreType.DMA,
    ],
    compiler_params=pltpu.CompilerParams(
        needs_layout_passes=False, use_tc_tiling_on_sc=False),
)
def _sc_call(didx_hbm, aidx_hbm, dtab_hbm, atab_hbm, out_hbm,
             didx_v, aidx_v, drows_v, arows_v, out_v, sem):
    _sc_body(didx_hbm, aidx_hbm, dtab_hbm, atab_hbm, out_hbm,
             didx_v, aidx_v, drows_v, arows_v, out_v, sem)


@jax.jit
def kernel(drug_idx, adr_idx, drug_embeddings, adr_embeddings):
    didx2 = drug_idx.reshape(NW * NCHUNK, CHUNK)
    aidx2 = adr_idx.reshape(NW * NCHUNK, CHUNK)
    return _sc_call(didx2, aidx2, drug_embeddings, adr_embeddings)
